# Initial kernel scaffold; baseline (speedup 1.0000x reference)
import jax
import jax.numpy as jnp
from jax.experimental import pallas as pl


def kernel(x, edge_index, W1, b1, W2, b2, W3, b3, g1, be1, g2, be2):
    raise NotImplementedError("write your pallas kernel here")


# speedup vs baseline: 10.9236x; 10.9236x over previous
import functools

import jax
import jax.numpy as jnp
from jax import lax
from jax.experimental import pallas as pl
from jax.experimental.pallas import tpu as pltpu
from jax.experimental.pallas import tpu_sc as plsc

_N = 10000
_NPAD = 10240
_E = 320000
_CHUNK = 128
_NCHUNKS = _E // _CHUNK
_NC = 2
_NS = 16
_NW = _NC * _NS
_RPT = _NPAD // _NS


def _sc_mesh():
    return plsc.VectorSubcoreMesh(core_axis_name="c", subcore_axis_name="s")


def _segsum(xs, eidx, feat):
    @functools.partial(
        pl.kernel,
        out_type=jax.ShapeDtypeStruct((_NC, _NPAD, feat), jnp.float32),
        mesh=_sc_mesh(),
        scratch_types=[
            pltpu.VMEM((2, _CHUNK), jnp.int32),
            pltpu.VMEM((_CHUNK, feat), jnp.float32),
            pltpu.VMEM_SHARED((_NPAD, feat), jnp.float32),
            pltpu.SemaphoreType.DMA,
        ],
    )
    def k(xs_hbm, eidx_hbm, zeros_hbm, out_hbm, idx_v, rows_v, acc, sem):
        c = lax.axis_index("c")
        s = lax.axis_index("s")
        wid = s * _NC + c
        base = s * _RPT
        pltpu.sync_copy(zeros_hbm.at[pl.ds(base, _RPT)], acc.at[pl.ds(base, _RPT)])
        plsc.subcore_barrier()

        @pl.loop(wid, _NCHUNKS, step=_NW)
        def _(i):
            pltpu.sync_copy(eidx_hbm.at[0, i], idx_v.at[0])
            pltpu.sync_copy(eidx_hbm.at[1, i], idx_v.at[1])
            pltpu.async_copy(xs_hbm.at[idx_v.at[0]], rows_v, sem).wait()
            pltpu.sync_copy(rows_v, acc.at[idx_v.at[1]], add=True)

        plsc.subcore_barrier()
        pltpu.sync_copy(acc.at[pl.ds(base, _RPT)], out_hbm.at[c, pl.ds(base, _RPT)])

    return k(xs, eidx, jnp.zeros((_NPAD, feat), jnp.float32))


def _degcount(eidx):
    @functools.partial(
        pl.kernel,
        out_type=jax.ShapeDtypeStruct((_NC, _NPAD, 128), jnp.float32),
        mesh=_sc_mesh(),
        scratch_types=[
            pltpu.VMEM((2, _CHUNK), jnp.int32),
            pltpu.VMEM((_CHUNK, 128), jnp.float32),
            pltpu.VMEM_SHARED((_NPAD, 128), jnp.float32),
        ],
    )
    def k(eidx_hbm, ones_hbm, zeros_hbm, out_hbm, idx_v, ones_v, acc):
        c = lax.axis_index("c")
        s = lax.axis_index("s")
        wid = s * _NC + c
        base = s * _RPT
        pltpu.sync_copy(ones_hbm, ones_v)
        pltpu.sync_copy(zeros_hbm.at[pl.ds(base, _RPT)], acc.at[pl.ds(base, _RPT)])
        plsc.subcore_barrier()

        @pl.loop(wid, _NCHUNKS, step=_NW)
        def _(i):
            pltpu.sync_copy(eidx_hbm.at[0, i], idx_v.at[0])
            pltpu.sync_copy(ones_v, acc.at[idx_v.at[0]], add=True)

        plsc.subcore_barrier()
        pltpu.sync_copy(acc.at[pl.ds(base, _RPT)], out_hbm.at[c, pl.ds(base, _RPT)])

    return k(
        eidx,
        jnp.ones((_CHUNK, 128), jnp.float32),
        jnp.zeros((_NPAD, 128), jnp.float32),
    )


def _bn_lrelu(h, g, be):
    m = jnp.mean(h, axis=0, keepdims=True)
    v = jnp.mean(h * h, axis=0, keepdims=True) - m * m
    hn = g * (h - m) * lax.rsqrt(v + 1e-5) + be
    return jnp.where(hn > 0, hn, 0.01 * hn)


def _t1(x_ref, dg_ref, w10_ref, b1_ref, dinv_ref, xs_ref, p1_ref):
    deg = dg_ref[0, :_N, 0:1] + dg_ref[1, :_N, 0:1]
    dinv = jnp.where(deg > 0, lax.rsqrt(deg), 0.0)
    dinv_ref[...] = dinv
    xv = x_ref[...]
    xs_ref[...] = xv * dinv
    p1_ref[...] = (
        jnp.dot(xv, w10_ref[...], preferred_element_type=jnp.float32) + b1_ref[...]
    )


def _t2(p1_ref, acc_ref, dinv_ref, w11_ref, g1_ref, be1_ref, w20_ref, b2_ref,
        p2_ref, y2_ref):
    dinv = dinv_ref[...]
    tx = -(acc_ref[0, :_N, :] + acc_ref[1, :_N, :]) * dinv
    h = p1_ref[...] + jnp.dot(tx, w11_ref[...], preferred_element_type=jnp.float32)
    h = _bn_lrelu(h, g1_ref[...], be1_ref[...])
    p2_ref[...] = (
        jnp.dot(h, w20_ref[...], preferred_element_type=jnp.float32) + b2_ref[...]
    )
    y2_ref[...] = h * dinv


def _t3(p2_ref, acc_ref, dinv_ref, w21_ref, g2_ref, be2_ref, w30_ref, b3_ref,
        p3_ref, y3_ref):
    dinv = dinv_ref[...]
    tx = -(acc_ref[0, :_N, :] + acc_ref[1, :_N, :]) * dinv
    h2 = p2_ref[...] + jnp.dot(tx, w21_ref[...], preferred_element_type=jnp.float32)
    z = _bn_lrelu(h2, g2_ref[...], be2_ref[...])
    p3_ref[...] = (
        jnp.dot(z, w30_ref[...], preferred_element_type=jnp.float32) + b3_ref[...]
    )
    y3_ref[:, 0:64] = z * dinv
    y3_ref[:, 64:128] = jnp.zeros((_N, 64), jnp.float32)


def _t4(p3_ref, acc_ref, dinv_ref, w31_ref, out_ref):
    tx = -(acc_ref[0, :_N, 0:64] + acc_ref[1, :_N, 0:64]) * dinv_ref[...]
    out_ref[...] = p3_ref[...] + jnp.dot(
        tx, w31_ref[...], preferred_element_type=jnp.float32
    )


def _f32(shape):
    return jax.ShapeDtypeStruct(shape, jnp.float32)


def kernel(x, edge_index, W1, b1, W2, b2, W3, b3, g1, be1, g2, be2):
    C, H, Hh = 128, 128, 64
    eidx = edge_index.reshape(2, _NCHUNKS, _CHUNK)

    dg = _degcount(eidx)

    dinv, xs, p1 = pl.pallas_call(
        _t1, out_shape=(_f32((_N, 1)), _f32((_N, C)), _f32((_N, H))),
    )(x, dg, W1[0], b1.reshape(1, H))

    acc1 = _segsum(xs, eidx, C)

    p2, y2 = pl.pallas_call(
        _t2, out_shape=(_f32((_N, Hh)), _f32((_N, H))),
    )(p1, acc1, dinv, W1[1], g1.reshape(1, H), be1.reshape(1, H),
      W2[0], b2.reshape(1, Hh))

    acc2 = _segsum(y2, eidx, H)

    p3, y3 = pl.pallas_call(
        _t3, out_shape=(_f32((_N, C)), _f32((_N, H))),
    )(p2, acc2, dinv, W2[1], g2.reshape(1, Hh), be2.reshape(1, Hh),
      W3[0], b3.reshape(1, C))

    acc3 = _segsum(y3, eidx, H)

    out = pl.pallas_call(
        _t4, out_shape=_f32((_N, C)),
    )(p3, acc3, dinv, W3[1])

    return out
